# Initial kernel scaffold; baseline (speedup 1.0000x reference)
import jax
import jax.numpy as jnp
from jax.experimental import pallas as pl


def kernel(h, edge_index, pos_enc, WQ, WK, WV, WO, bO, W1, b1, W2, b2, g1, be1, g2, be2):
    raise NotImplementedError("write your pallas kernel here")


# speedup vs baseline: 12.0959x; 12.0959x over previous
import functools

import jax
import jax.numpy as jnp
import numpy as np
from jax import lax
from jax.experimental import pallas as pl
from jax.experimental.pallas import tpu as pltpu
from jax.experimental.pallas import tpu_sc as plsc

N = 10000
E = 320000
D = 128
H = 8
DH = 16

NC = 2
NS = 16
NW = NC * NS
EPW = E // NW
C = 80
NCHUNK = EPW // C
G = C // 16
NP = 10240
RPT = NP // NS
ZR = 128


def _qkv_body(h_ref, wq_ref, wk_ref, wv_ref, q_out, k_out, v_out):
    x = h_ref[...]
    dn = (((1,), (1,)), ((), ()))
    q_out[...] = lax.dot_general(x, wq_ref[...], dn,
                                 preferred_element_type=jnp.float32)
    k_out[...] = lax.dot_general(x, wk_ref[...], dn,
                                 preferred_element_type=jnp.float32)
    v_out[...] = lax.dot_general(x, wv_ref[...], dn,
                                 preferred_element_type=jnp.float32)


def _edge_body(q_hbm, k_hbm, v_hbm, src_hbm, dst_hbm,
               wv_out, z_out,
               k_buf, q_buf, src_i, dst_i, wv_o, z_o,
               wv_acc, z_acc, sem0, sem1):
    cid = lax.axis_index("c")
    sid = lax.axis_index("s")
    wid = sid * NC + cid

    def zrow(r, _):
        for hh in range(8):
            wv_o[r, pl.ds(hh * 16, 16)] = jnp.zeros((16,), jnp.float32)
        z_o[r, pl.ds(0, 16)] = jnp.zeros((16,), jnp.float32)
        return 0
    lax.fori_loop(0, C, zrow, 0)
    for j in range(RPT // C):
        base = sid * RPT + j * C
        pltpu.sync_copy(wv_o, wv_acc.at[pl.ds(base, C)])
        pltpu.sync_copy(z_o, z_acc.at[pl.ds(base, C)])
    plsc.subcore_barrier()

    def chunk_body(ci, _):
        base = wid * EPW + ci * C
        pltpu.sync_copy(src_hbm.at[pl.ds(base, C)], src_i)
        pltpu.sync_copy(dst_hbm.at[pl.ds(base, C)], dst_i)
        ck = pltpu.async_copy(k_hbm.at[src_i], k_buf, sem0)
        cq = pltpu.async_copy(q_hbm.at[dst_i], q_buf, sem1)
        ck.wait()
        cq.wait()

        def score_body(g, _):
            ev = g * 16 + lax.iota(jnp.int32, 16)
            for h in range(H):
                acc = jnp.zeros((16,), jnp.float32)
                for d in range(DH):
                    cvec = jnp.full((16,), h * 16 + d, jnp.int32)
                    kv = plsc.load_gather(k_buf, [ev, cvec])
                    qv = plsc.load_gather(q_buf, [ev, cvec])
                    acc = acc + kv * qv
                sh = jnp.exp(jnp.clip(acc * 0.25, -5.0, 5.0))
                plsc.store_scatter(z_o, [ev, jnp.full((16,), h, jnp.int32)], sh)
            return 0
        lax.fori_loop(0, G, score_body, 0)

        cv = pltpu.async_copy(v_hbm.at[src_i], k_buf, sem0)
        cv.wait()

        def wv_body(g, _):
            ev = g * 16 + lax.iota(jnp.int32, 16)
            for h in range(H):
                sh = plsc.load_gather(z_o, [ev, jnp.full((16,), h, jnp.int32)])
                for d in range(DH):
                    cvec = jnp.full((16,), h * 16 + d, jnp.int32)
                    vv = plsc.load_gather(k_buf, [ev, cvec])
                    plsc.store_scatter(wv_o, [ev, cvec], vv * sh)
            return 0
        lax.fori_loop(0, G, wv_body, 0)

        pltpu.sync_copy(wv_o, wv_acc.at[dst_i], add=True)
        pltpu.sync_copy(z_o, z_acc.at[dst_i], add=True)
        return 0
    lax.fori_loop(0, NCHUNK, chunk_body, 0)

    plsc.subcore_barrier()

    for j in range(RPT // C):
        base = sid * RPT + j * C
        pltpu.sync_copy(wv_acc.at[pl.ds(base, C)],
                        wv_out.at[cid, pl.ds(base, C)])
        pltpu.sync_copy(z_acc.at[pl.ds(base, C)],
                        z_out.at[cid, pl.ds(base, C)])


_edge_kernel = functools.partial(
    pl.kernel,
    out_type=[jax.ShapeDtypeStruct((NC, NP, D), jnp.float32),
              jax.ShapeDtypeStruct((NC, NP, 16), jnp.float32)],
    mesh=plsc.VectorSubcoreMesh(core_axis_name="c", subcore_axis_name="s"),
    compiler_params=pltpu.CompilerParams(needs_layout_passes=False,
                                         use_tc_tiling_on_sc=False),
    scratch_types=[
        pltpu.VMEM((C, D), jnp.float32),
        pltpu.VMEM((C, D), jnp.float32),
        pltpu.VMEM((C,), jnp.int32),
        pltpu.VMEM((C,), jnp.int32),
        pltpu.VMEM((C, D), jnp.float32),
        pltpu.VMEM((C, 16), jnp.float32),
        pltpu.VMEM_SHARED((NP, D), jnp.float32),
        pltpu.VMEM_SHARED((NP, 16), jnp.float32),
        pltpu.SemaphoreType.DMA,
        pltpu.SemaphoreType.DMA,
    ],
)(_edge_body)


def _post_body(h_ref, wvp_ref, zp_ref, s_ref, wo_ref, bo_ref,
               w1_ref, b1_ref, w2_ref, b2_ref,
               g1_ref, be1_ref, g2_ref, be2_ref, out_ref):
    wv = wvp_ref[0, 0:N] + wvp_ref[1, 0:N]
    z = zp_ref[0, 0:N, 0:8] + zp_ref[1, 0:N, 0:8]
    dn = (((1,), (1,)), ((), ()))
    dn0 = (((1,), (0,)), ((), ()))
    zx = lax.dot_general(1.0 / z, s_ref[...], dn0,
                         preferred_element_type=jnp.float32)
    head = wv * zx
    hh = lax.dot_general(head, wo_ref[...], dn,
                         preferred_element_type=jnp.float32) + bo_ref[...]
    hh = h_ref[...] + hh
    mu = jnp.mean(hh, axis=0)
    var = jnp.mean((hh - mu) ** 2, axis=0)
    hh = (hh - mu) * lax.rsqrt(var + 1e-5) * g1_ref[...] + be1_ref[...]
    f = lax.dot_general(hh, w1_ref[...], dn,
                        preferred_element_type=jnp.float32) + b1_ref[...]
    f = jnp.maximum(f, 0.0)
    f = lax.dot_general(f, w2_ref[...], dn,
                        preferred_element_type=jnp.float32) + b2_ref[...]
    hh = hh + f
    mu2 = jnp.mean(hh, axis=0)
    var2 = jnp.mean((hh - mu2) ** 2, axis=0)
    out_ref[...] = ((hh - mu2) * lax.rsqrt(var2 + 1e-5) * g2_ref[...]
                    + be2_ref[...])


_S = np.repeat(np.eye(H, dtype=np.float32), DH, axis=1)


def kernel(h, edge_index, pos_enc, WQ, WK, WV, WO, bO, W1, b1, W2, b2,
           g1, be1, g2, be2):
    src = edge_index[0].astype(jnp.int32)
    dst = edge_index[1].astype(jnp.int32)

    q, k, v = pl.pallas_call(
        _qkv_body,
        out_shape=[jax.ShapeDtypeStruct((N, D), jnp.float32)] * 3,
    )(h, WQ, WK, WV)

    wvp, zp = _edge_kernel(q, k, v, src, dst)

    out = pl.pallas_call(
        _post_body,
        out_shape=jax.ShapeDtypeStruct((N, D), jnp.float32),
    )(h, wvp, zp, jnp.asarray(_S), WO, bO, W1, b1, W2, b2, g1, be1, g2, be2)
    return out


# speedup vs baseline: 35.2931x; 2.9178x over previous
import functools

import jax
import jax.numpy as jnp
import numpy as np
from jax import lax
from jax.experimental import pallas as pl
from jax.experimental.pallas import tpu as pltpu
from jax.experimental.pallas import tpu_sc as plsc

N = 10000
E = 320000
D = 128
H = 8
DH = 16

NC = 2
NS = 16
NW = NC * NS
EPW = E // NW
C = 80
NCHUNK = EPW // C
G = C // 16
NP = 10240
RPT = NP // NS
ZR = 128


def _qkv_body(h_ref, wq_ref, wk_ref, wv_ref, q_out, k_out, v_out):
    x = h_ref[...]
    dn = (((1,), (1,)), ((), ()))
    q_out[...] = lax.dot_general(x, wq_ref[...], dn,
                                 preferred_element_type=jnp.float32)
    k_out[...] = lax.dot_general(x, wk_ref[...], dn,
                                 preferred_element_type=jnp.float32)
    v_out[...] = lax.dot_general(x, wv_ref[...], dn,
                                 preferred_element_type=jnp.float32)


def _edge_body(q_hbm, k_hbm, v_hbm, src_hbm, dst_hbm,
               wv_out, z_out,
               k_buf, q_buf, src_i, dst_i, wv_o, z_o, z_s,
               wv_acc, z_acc, sem0, sem1):
    cid = lax.axis_index("c")
    sid = lax.axis_index("s")
    wid = sid * NC + cid

    def zrow(r, _):
        for hh in range(8):
            wv_o[r, pl.ds(hh * 16, 16)] = jnp.zeros((16,), jnp.float32)
        z_o[r, pl.ds(0, 16)] = jnp.zeros((16,), jnp.float32)
        return 0
    lax.fori_loop(0, C, zrow, 0)
    for j in range(RPT // C):
        base = sid * RPT + j * C
        pltpu.sync_copy(wv_o, wv_acc.at[pl.ds(base, C)])
        pltpu.sync_copy(z_o, z_acc.at[pl.ds(base, C)])
    plsc.subcore_barrier()

    def chunk_body(ci, _):
        base = wid * EPW + ci * C
        pltpu.sync_copy(src_hbm.at[pl.ds(base, C)], src_i)
        pltpu.sync_copy(dst_hbm.at[pl.ds(base, C)], dst_i)
        ck = pltpu.async_copy(k_hbm.at[src_i], k_buf, sem0)
        cq = pltpu.async_copy(q_hbm.at[dst_i], q_buf, sem1)
        ck.wait()
        cq.wait()

        lv = lax.iota(jnp.int32, 16)

        def score_body(g, _):
            ev = g * 16 + lv
            for h in range(H):
                acc0 = jnp.zeros((16,), jnp.float32)
                acc1 = jnp.zeros((16,), jnp.float32)
                for i in range(DH):
                    cvec = h * 16 + ((lv + i) & 15)
                    kv = plsc.load_gather(k_buf, [ev, cvec])
                    qv = plsc.load_gather(q_buf, [ev, cvec])
                    if i % 2 == 0:
                        acc0 = acc0 + kv * qv
                    else:
                        acc1 = acc1 + kv * qv
                sh = jnp.exp(jnp.clip((acc0 + acc1) * 0.25, -5.0, 5.0))
                plsc.store_scatter(z_s, [ev, jnp.full((16,), h, jnp.int32)], sh)
            return 0
        lax.fori_loop(0, G, score_body, 0)

        def zcopy_body(r, _):
            row = plsc.load_gather(z_s, [jnp.full((16,), r, jnp.int32), lv])
            z_o[r, pl.ds(0, 16)] = row
            return 0
        lax.fori_loop(0, C, zcopy_body, 0)

        cv = pltpu.async_copy(v_hbm.at[src_i], k_buf, sem0)
        cv.wait()

        def wv_body(g, _):
            ev = g * 16 + lv
            for h in range(H):
                sh = plsc.load_gather(z_s, [ev, jnp.full((16,), h, jnp.int32)])
                for i in range(DH):
                    cvec = h * 16 + ((lv + i) & 15)
                    vv = plsc.load_gather(k_buf, [ev, cvec])
                    plsc.store_scatter(wv_o, [ev, cvec], vv * sh)
            return 0
        lax.fori_loop(0, G, wv_body, 0)

        pltpu.sync_copy(wv_o, wv_acc.at[dst_i], add=True)
        pltpu.sync_copy(z_o, z_acc.at[dst_i], add=True)
        return 0
    lax.fori_loop(0, NCHUNK, chunk_body, 0)

    plsc.subcore_barrier()

    for j in range(RPT // C):
        base = sid * RPT + j * C
        pltpu.sync_copy(wv_acc.at[pl.ds(base, C)],
                        wv_out.at[cid, pl.ds(base, C)])
        pltpu.sync_copy(z_acc.at[pl.ds(base, C)],
                        z_out.at[cid, pl.ds(base, C)])


_edge_kernel = functools.partial(
    pl.kernel,
    out_type=[jax.ShapeDtypeStruct((NC, NP, D), jnp.float32),
              jax.ShapeDtypeStruct((NC, NP, 16), jnp.float32)],
    mesh=plsc.VectorSubcoreMesh(core_axis_name="c", subcore_axis_name="s"),
    compiler_params=pltpu.CompilerParams(needs_layout_passes=False,
                                         use_tc_tiling_on_sc=False),
    scratch_types=[
        pltpu.VMEM((C, D), jnp.float32),
        pltpu.VMEM((C, D), jnp.float32),
        pltpu.VMEM((C,), jnp.int32),
        pltpu.VMEM((C,), jnp.int32),
        pltpu.VMEM((C, D), jnp.float32),
        pltpu.VMEM((C, 16), jnp.float32),
        pltpu.VMEM((C, 17), jnp.float32),
        pltpu.VMEM_SHARED((NP, D), jnp.float32),
        pltpu.VMEM_SHARED((NP, 16), jnp.float32),
        pltpu.SemaphoreType.DMA,
        pltpu.SemaphoreType.DMA,
    ],
)(_edge_body)


def _post_body(h_ref, wvp_ref, zp_ref, s_ref, wo_ref, bo_ref,
               w1_ref, b1_ref, w2_ref, b2_ref,
               g1_ref, be1_ref, g2_ref, be2_ref, out_ref):
    wv = wvp_ref[0, 0:N] + wvp_ref[1, 0:N]
    z = zp_ref[0, 0:N, 0:8] + zp_ref[1, 0:N, 0:8]
    dn = (((1,), (1,)), ((), ()))
    dn0 = (((1,), (0,)), ((), ()))
    zx = lax.dot_general(1.0 / z, s_ref[...], dn0,
                         preferred_element_type=jnp.float32)
    head = wv * zx
    hh = lax.dot_general(head, wo_ref[...], dn,
                         preferred_element_type=jnp.float32) + bo_ref[...]
    hh = h_ref[...] + hh
    mu = jnp.mean(hh, axis=0)
    var = jnp.mean((hh - mu) ** 2, axis=0)
    hh = (hh - mu) * lax.rsqrt(var + 1e-5) * g1_ref[...] + be1_ref[...]
    f = lax.dot_general(hh, w1_ref[...], dn,
                        preferred_element_type=jnp.float32) + b1_ref[...]
    f = jnp.maximum(f, 0.0)
    f = lax.dot_general(f, w2_ref[...], dn,
                        preferred_element_type=jnp.float32) + b2_ref[...]
    hh = hh + f
    mu2 = jnp.mean(hh, axis=0)
    var2 = jnp.mean((hh - mu2) ** 2, axis=0)
    out_ref[...] = ((hh - mu2) * lax.rsqrt(var2 + 1e-5) * g2_ref[...]
                    + be2_ref[...])


_S = np.repeat(np.eye(H, dtype=np.float32), DH, axis=1)


def kernel(h, edge_index, pos_enc, WQ, WK, WV, WO, bO, W1, b1, W2, b2,
           g1, be1, g2, be2):
    src = edge_index[0].astype(jnp.int32)
    dst = edge_index[1].astype(jnp.int32)

    q, k, v = pl.pallas_call(
        _qkv_body,
        out_shape=[jax.ShapeDtypeStruct((N, D), jnp.float32)] * 3,
    )(h, WQ, WK, WV)

    wvp, zp = _edge_kernel(q, k, v, src, dst)

    out = pl.pallas_call(
        _post_body,
        out_shape=jax.ShapeDtypeStruct((N, D), jnp.float32),
    )(h, wvp, zp, jnp.asarray(_S), WO, bO, W1, b1, W2, b2, g1, be1, g2, be2)
    return out


# speedup vs baseline: 40.1515x; 1.1377x over previous
import functools

import jax
import jax.numpy as jnp
import numpy as np
from jax import lax
from jax.experimental import pallas as pl
from jax.experimental.pallas import tpu as pltpu
from jax.experimental.pallas import tpu_sc as plsc

N = 10000
E = 320000
D = 128
H = 8
DH = 16

NC = 2
NS = 16
NW = NC * NS
EPW = E // NW
C = 80
NCHUNK = EPW // C
G = C // 16
NP = 10240
RPT = NP // NS
ZR = 128


def _qkv_body(h_ref, wq_ref, wk_ref, wv_ref, q_out, k_out, v_out):
    x = h_ref[...]
    dn = (((1,), (1,)), ((), ()))
    q_out[...] = lax.dot_general(x, wq_ref[...], dn,
                                 preferred_element_type=jnp.float32)
    k_out[...] = lax.dot_general(x, wk_ref[...], dn,
                                 preferred_element_type=jnp.float32)
    v_out[...] = lax.dot_general(x, wv_ref[...], dn,
                                 preferred_element_type=jnp.float32)


def _edge_body(q_hbm, k_hbm, v_hbm, ei_hbm,
               wv_out, z_out,
               k_buf, q_buf, v_buf, idx2, z_o, z_s,
               wv_acc, z_acc, sem0, sem1, sem2):
    cid = lax.axis_index("c")
    sid = lax.axis_index("s")
    wid = sid * NC + cid
    lv = lax.iota(jnp.int32, 16)

    def zrow(r, _):
        for hh in range(8):
            v_buf[r, pl.ds(hh * 16, 16)] = jnp.zeros((16,), jnp.float32)
        z_o[r, pl.ds(0, 16)] = jnp.zeros((16,), jnp.float32)
        return 0
    lax.fori_loop(0, C, zrow, 0)
    for j in range(RPT // C):
        base = sid * RPT + j * C
        pltpu.sync_copy(v_buf, wv_acc.at[pl.ds(base, C)])
        pltpu.sync_copy(z_o, z_acc.at[pl.ds(base, C)])
    plsc.subcore_barrier()

    def chunk_body(ci, _):
        base = wid * EPW + ci * C
        pltpu.sync_copy(ei_hbm.at[:, pl.ds(base, C)], idx2)
        src_i = idx2.at[0]
        dst_i = idx2.at[1]
        ck = pltpu.async_copy(k_hbm.at[src_i], k_buf, sem0)
        cq = pltpu.async_copy(q_hbm.at[dst_i], q_buf, sem1)
        cv = pltpu.async_copy(v_hbm.at[src_i], v_buf, sem2)
        ck.wait()
        cq.wait()

        def score_body(g, _):
            ev = g * 16 + lv
            for h in range(H):
                acc0 = jnp.zeros((16,), jnp.float32)
                acc1 = jnp.zeros((16,), jnp.float32)
                for i in range(DH):
                    cvec = h * 16 + ((lv + i) & 15)
                    kv = plsc.load_gather(k_buf, [ev, cvec])
                    qv = plsc.load_gather(q_buf, [ev, cvec])
                    if i % 2 == 0:
                        acc0 = acc0 + kv * qv
                    else:
                        acc1 = acc1 + kv * qv
                sh = jnp.exp(jnp.clip((acc0 + acc1) * 0.25, -5.0, 5.0))
                plsc.store_scatter(z_s, [ev, jnp.full((16,), h, jnp.int32)], sh)
            return 0
        lax.fori_loop(0, G, score_body, 0)

        def zcopy_body(r, _):
            row = plsc.load_gather(z_s, [jnp.full((16,), r, jnp.int32), lv])
            z_o[r, pl.ds(0, 16)] = row
            return 0
        lax.fori_loop(0, C, zcopy_body, 0)

        cv.wait()

        def wv_body(g, _):
            ev = g * 16 + lv
            for h in range(H):
                sh = plsc.load_gather(z_s, [ev, jnp.full((16,), h, jnp.int32)])
                for i in range(DH):
                    cvec = h * 16 + ((lv + i) & 15)
                    vv = plsc.load_gather(v_buf, [ev, cvec])
                    plsc.store_scatter(v_buf, [ev, cvec], vv * sh)
            return 0
        lax.fori_loop(0, G, wv_body, 0)

        sa = pltpu.async_copy(v_buf, wv_acc.at[dst_i], sem0, add=True)
        sz = pltpu.async_copy(z_o, z_acc.at[dst_i], sem1, add=True)
        sa.wait()
        sz.wait()
        return 0
    lax.fori_loop(0, NCHUNK, chunk_body, 0)

    plsc.subcore_barrier()

    for j in range(RPT // C):
        base = sid * RPT + j * C
        pltpu.sync_copy(wv_acc.at[pl.ds(base, C)],
                        wv_out.at[cid, pl.ds(base, C)])
        pltpu.sync_copy(z_acc.at[pl.ds(base, C)],
                        z_out.at[cid, pl.ds(base, C)])


_edge_kernel = functools.partial(
    pl.kernel,
    out_type=[jax.ShapeDtypeStruct((NC, NP, D), jnp.float32),
              jax.ShapeDtypeStruct((NC, NP, 16), jnp.float32)],
    mesh=plsc.VectorSubcoreMesh(core_axis_name="c", subcore_axis_name="s"),
    compiler_params=pltpu.CompilerParams(needs_layout_passes=False,
                                         use_tc_tiling_on_sc=False),
    scratch_types=[
        pltpu.VMEM((C, D), jnp.float32),
        pltpu.VMEM((C, D), jnp.float32),
        pltpu.VMEM((C, D), jnp.float32),
        pltpu.VMEM((2, C), jnp.int32),
        pltpu.VMEM((C, 16), jnp.float32),
        pltpu.VMEM((C, 17), jnp.float32),
        pltpu.VMEM_SHARED((NP, D), jnp.float32),
        pltpu.VMEM_SHARED((NP, 16), jnp.float32),
        pltpu.SemaphoreType.DMA,
        pltpu.SemaphoreType.DMA,
        pltpu.SemaphoreType.DMA,
    ],
)(_edge_body)


def _post_body(h_ref, wvp_ref, zp_ref, s_ref, wo_ref, bo_ref,
               w1_ref, b1_ref, w2_ref, b2_ref,
               g1_ref, be1_ref, g2_ref, be2_ref, out_ref):
    wv = wvp_ref[0, 0:N] + wvp_ref[1, 0:N]
    z = zp_ref[0, 0:N, 0:8] + zp_ref[1, 0:N, 0:8]
    dn = (((1,), (1,)), ((), ()))
    dn0 = (((1,), (0,)), ((), ()))
    zx = lax.dot_general(1.0 / z, s_ref[...], dn0,
                         preferred_element_type=jnp.float32)
    head = wv * zx
    hh = lax.dot_general(head, wo_ref[...], dn,
                         preferred_element_type=jnp.float32) + bo_ref[...]
    hh = h_ref[...] + hh
    mu = jnp.mean(hh, axis=0)
    var = jnp.mean((hh - mu) ** 2, axis=0)
    hh = (hh - mu) * lax.rsqrt(var + 1e-5) * g1_ref[...] + be1_ref[...]
    f = lax.dot_general(hh, w1_ref[...], dn,
                        preferred_element_type=jnp.float32) + b1_ref[...]
    f = jnp.maximum(f, 0.0)
    f = lax.dot_general(f, w2_ref[...], dn,
                        preferred_element_type=jnp.float32) + b2_ref[...]
    hh = hh + f
    mu2 = jnp.mean(hh, axis=0)
    var2 = jnp.mean((hh - mu2) ** 2, axis=0)
    out_ref[...] = ((hh - mu2) * lax.rsqrt(var2 + 1e-5) * g2_ref[...]
                    + be2_ref[...])


_S = np.repeat(np.eye(H, dtype=np.float32), DH, axis=1)


def kernel(h, edge_index, pos_enc, WQ, WK, WV, WO, bO, W1, b1, W2, b2,
           g1, be1, g2, be2):
    ei = edge_index.astype(jnp.int32)

    q, k, v = pl.pallas_call(
        _qkv_body,
        out_shape=[jax.ShapeDtypeStruct((N, D), jnp.float32)] * 3,
    )(h, WQ, WK, WV)

    wvp, zp = _edge_kernel(q, k, v, ei)

    out = pl.pallas_call(
        _post_body,
        out_shape=jax.ShapeDtypeStruct((N, D), jnp.float32),
    )(h, wvp, zp, jnp.asarray(_S), WO, bO, W1, b1, W2, b2, g1, be1, g2, be2)
    return out
